# baseline (speedup 1.0000x reference)
import functools

import jax
import jax.numpy as jnp
from jax import lax
from jax.experimental import pallas as pl
from jax.experimental.pallas import tpu as pltpu
from jax.experimental.pallas import tpu_sc as plsc

VOCAB = 100000
DIM = 128
RANK = 64
LORA_SCALE = 1.0 / RANK


MERGE_BLK = 10000


def _merge_body(w_ref, a_ref, b_ref, out_ref):
    lora = lax.dot_general(
        a_ref[...].astype(jnp.bfloat16), b_ref[...],
        (((1,), (1,)), ((), ())),
        preferred_element_type=jnp.float32,
    )
    out_ref[...] = w_ref[...] + lora * LORA_SCALE


def _merge(W_src, A, B):
    return pl.pallas_call(
        _merge_body,
        grid=(VOCAB // MERGE_BLK,),
        in_specs=[
            pl.BlockSpec((MERGE_BLK, DIM), lambda i: (i, 0)),
            pl.BlockSpec((MERGE_BLK, RANK), lambda i: (i, 0)),
            pl.BlockSpec((DIM, RANK), lambda i: (0, 0)),
        ],
        out_specs=pl.BlockSpec((MERGE_BLK, DIM), lambda i: (i, 0)),
        out_shape=jax.ShapeDtypeStruct((VOCAB, DIM), jnp.float32),
    )(W_src, A, B.astype(jnp.bfloat16))


NTOK = 4096 * 200
NC, NS = 2, 16
NW = NC * NS
CHUNK = 128
NCHUNK = NTOK // (NW * CHUNK)
GCH = 3
NGRP = NCHUNK // GCH
NREM = NCHUNK - NGRP * GCH


def _gather_sc_body(idx_hbm, tab_hbm, out_hbm, idx_v, rows, g0, g1, s0, s1):
    gs, ss = (g0, g1), (s0, s1)
    wid = lax.axis_index("s") * NC + lax.axis_index("c")
    base = wid * NCHUNK
    pltpu.sync_copy(idx_hbm.at[pl.ds(base, NCHUNK)], idx_v)

    def gather(slab, c, j):
        return pltpu.make_async_copy(
            tab_hbm.at[idx_v.at[c]],
            rows.at[slab].at[pl.ds(j * CHUNK, CHUNK)], gs[slab])

    def store(slab, g):
        return pltpu.make_async_copy(
            rows.at[slab],
            out_hbm.at[pl.ds((base + g * GCH) * CHUNK, GCH * CHUNK)],
            ss[slab])

    def fill(slab, g):
        for j in range(GCH):
            gather(slab, g * GCH + j, j).start()

    fill(0, 0)

    def pair(p, carry):
        for s in (0, 1):
            g = p * 2 + s
            for j in range(GCH):
                gather(s, g * GCH + j, j).wait()
            store(s, g).start()
            o = 1 - s

            @pl.when(g >= 1)
            def _():
                store(o, g - 1).wait()

            @pl.when(g <= NGRP - 2)
            def _():
                fill(o, g + 1)

        return carry

    lax.fori_loop(0, NGRP // 2, pair, 0)

    store(1, NGRP - 1).wait()

    for j in range(NREM):
        gather(0, NGRP * GCH + j, j).start()
    for j in range(NREM):
        gather(0, NGRP * GCH + j, j).wait()
    pltpu.make_async_copy(
        rows.at[0].at[pl.ds(0, NREM * CHUNK)],
        out_hbm.at[pl.ds((base + NGRP * GCH) * CHUNK, NREM * CHUNK)],
        ss[0]).start()
    pltpu.make_async_copy(
        rows.at[0].at[pl.ds(0, NREM * CHUNK)],
        out_hbm.at[pl.ds((base + NGRP * GCH) * CHUNK, NREM * CHUNK)],
        ss[0]).wait()


@functools.partial(
    pl.kernel,
    out_type=jax.ShapeDtypeStruct((NTOK, DIM), jnp.float32),
    mesh=plsc.VectorSubcoreMesh(
        core_axis_name="c", subcore_axis_name="s",
        num_cores=NC, num_subcores=NS),
    scratch_types=[
        pltpu.VMEM((NCHUNK, CHUNK), jnp.int32),
        pltpu.VMEM((2, GCH * CHUNK, DIM), jnp.float32),
    ] + [pltpu.SemaphoreType.DMA] * 4,
)
def _gather_sc(idx_hbm, tab_hbm, out_hbm, idx_v, rows, g0, g1, s0, s1):
    _gather_sc_body(idx_hbm, tab_hbm, out_hbm, idx_v, rows, g0, g1, s0, s1)


def kernel(x, W_src, A, B):
    merged = _merge(W_src, A, B)
    idx = x.reshape(NW * NCHUNK, CHUNK).astype(jnp.int32)
    out = _gather_sc(idx, merged)
    return out.reshape(x.shape[0], x.shape[1], DIM)


# speedup vs baseline: 1.0107x; 1.0107x over previous
import functools

import jax
import jax.numpy as jnp
from jax import lax
from jax.experimental import pallas as pl
from jax.experimental.pallas import tpu as pltpu
from jax.experimental.pallas import tpu_sc as plsc

VOCAB = 100000
DIM = 128
RANK = 64
LORA_SCALE = 1.0 / RANK


MERGE_BLK = 10000


def _merge_body(w_ref, a_ref, b_ref, out_ref):
    lora = lax.dot_general(
        a_ref[...], b_ref[...],
        (((1,), (1,)), ((), ())),
        preferred_element_type=jnp.float32,
    )
    out_ref[...] = w_ref[...] + lora * LORA_SCALE


def _merge(W_src, A, B):
    return pl.pallas_call(
        _merge_body,
        grid=(VOCAB // MERGE_BLK,),
        in_specs=[
            pl.BlockSpec((MERGE_BLK, DIM), lambda i: (i, 0)),
            pl.BlockSpec((MERGE_BLK, RANK), lambda i: (i, 0)),
            pl.BlockSpec((DIM, RANK), lambda i: (0, 0)),
        ],
        out_specs=pl.BlockSpec((MERGE_BLK, DIM), lambda i: (i, 0)),
        out_shape=jax.ShapeDtypeStruct((VOCAB, DIM), jnp.float32),
    )(W_src, A.astype(jnp.bfloat16), B.astype(jnp.bfloat16))


NTOK = 4096 * 200
NC, NS = 2, 16
NW = NC * NS
CHUNK = 128
NCHUNK = NTOK // (NW * CHUNK)
GCH = 3
NGRP = NCHUNK // GCH
NREM = NCHUNK - NGRP * GCH


def _gather_sc_body(idx_hbm, tab_hbm, out_hbm, idx_v, rows, g0, g1, s0, s1):
    gs, ss = (g0, g1), (s0, s1)
    wid = lax.axis_index("s") * NC + lax.axis_index("c")
    base = wid * NCHUNK
    pltpu.sync_copy(idx_hbm.at[pl.ds(base, NCHUNK)], idx_v)

    def gather(slab, c, j):
        return pltpu.make_async_copy(
            tab_hbm.at[idx_v.at[c]],
            rows.at[slab].at[pl.ds(j * CHUNK, CHUNK)], gs[slab])

    def store(slab, g):
        return pltpu.make_async_copy(
            rows.at[slab],
            out_hbm.at[pl.ds((base + g * GCH) * CHUNK, GCH * CHUNK)],
            ss[slab])

    def fill(slab, g):
        for j in range(GCH):
            gather(slab, g * GCH + j, j).start()

    fill(0, 0)

    def pair(p, carry):
        for s in (0, 1):
            g = p * 2 + s
            for j in range(GCH):
                gather(s, g * GCH + j, j).wait()
            store(s, g).start()
            o = 1 - s

            @pl.when(g >= 1)
            def _():
                store(o, g - 1).wait()

            @pl.when(g <= NGRP - 2)
            def _():
                fill(o, g + 1)

        return carry

    lax.fori_loop(0, NGRP // 2, pair, 0)

    store(1, NGRP - 1).wait()

    for j in range(NREM):
        gather(0, NGRP * GCH + j, j).start()
    for j in range(NREM):
        gather(0, NGRP * GCH + j, j).wait()
    pltpu.make_async_copy(
        rows.at[0].at[pl.ds(0, NREM * CHUNK)],
        out_hbm.at[pl.ds((base + NGRP * GCH) * CHUNK, NREM * CHUNK)],
        ss[0]).start()
    pltpu.make_async_copy(
        rows.at[0].at[pl.ds(0, NREM * CHUNK)],
        out_hbm.at[pl.ds((base + NGRP * GCH) * CHUNK, NREM * CHUNK)],
        ss[0]).wait()


@functools.partial(
    pl.kernel,
    out_type=jax.ShapeDtypeStruct((NTOK, DIM), jnp.float32),
    mesh=plsc.VectorSubcoreMesh(
        core_axis_name="c", subcore_axis_name="s",
        num_cores=NC, num_subcores=NS),
    scratch_types=[
        pltpu.VMEM((NCHUNK, CHUNK), jnp.int32),
        pltpu.VMEM((2, GCH * CHUNK, DIM), jnp.float32),
    ] + [pltpu.SemaphoreType.DMA] * 4,
)
def _gather_sc(idx_hbm, tab_hbm, out_hbm, idx_v, rows, g0, g1, s0, s1):
    _gather_sc_body(idx_hbm, tab_hbm, out_hbm, idx_v, rows, g0, g1, s0, s1)


def kernel(x, W_src, A, B):
    merged = _merge(W_src, A, B)
    idx = x.reshape(NW * NCHUNK, CHUNK).astype(jnp.int32)
    out = _gather_sc(idx, merged)
    return out.reshape(x.shape[0], x.shape[1], DIM)


# speedup vs baseline: 1.0150x; 1.0042x over previous
import functools

import jax
import jax.numpy as jnp
from jax import lax
from jax.experimental import pallas as pl
from jax.experimental.pallas import tpu as pltpu
from jax.experimental.pallas import tpu_sc as plsc

VOCAB = 100000
DIM = 128
RANK = 64
LORA_SCALE = 1.0 / RANK


MERGE_BLK = 10000


def _merge_body(w_ref, a_ref, b_ref, out_ref):
    lora = lax.dot_general(
        a_ref[...], b_ref[...],
        (((1,), (1,)), ((), ())),
        preferred_element_type=jnp.float32,
    )
    out_ref[...] = w_ref[...] + lora * LORA_SCALE


def _merge(W_src, A, B):
    return pl.pallas_call(
        _merge_body,
        grid=(VOCAB // MERGE_BLK,),
        in_specs=[
            pl.BlockSpec((MERGE_BLK, DIM), lambda i: (i, 0)),
            pl.BlockSpec((MERGE_BLK, RANK), lambda i: (i, 0)),
            pl.BlockSpec((DIM, RANK), lambda i: (0, 0)),
        ],
        out_specs=pl.BlockSpec((MERGE_BLK, DIM), lambda i: (i, 0)),
        out_shape=jax.ShapeDtypeStruct((VOCAB, DIM), jnp.float32),
    )(W_src, A.astype(jnp.bfloat16), B.astype(jnp.bfloat16))


NTOK = 4096 * 200
NC, NS = 2, 16
NW = NC * NS
CHUNK = 128
NCHUNK = NTOK // (NW * CHUNK)
GCH = 2
NSLAB = 3
NGRP = NCHUNK // GCH
NREM = NCHUNK - NGRP * GCH


def _gather_sc_body(idx_hbm, tab_hbm, out_hbm, idx_v, rows, g0, g1, g2, s0, s1, s2):
    gs, ss = (g0, g1, g2), (s0, s1, s2)
    wid = lax.axis_index("s") * NC + lax.axis_index("c")
    base = wid * NCHUNK
    pltpu.sync_copy(idx_hbm.at[pl.ds(base, NCHUNK)], idx_v)

    def gather(slab, c, j):
        return pltpu.make_async_copy(
            tab_hbm.at[idx_v.at[c]],
            rows.at[slab].at[pl.ds(j * CHUNK, CHUNK)], gs[slab])

    def store(slab, g):
        return pltpu.make_async_copy(
            rows.at[slab],
            out_hbm.at[pl.ds((base + g * GCH) * CHUNK, GCH * CHUNK)],
            ss[slab])

    def fill(slab, g):
        for j in range(GCH):
            gather(slab, g * GCH + j, j).start()

    fill(0, 0)
    fill(1, 1)

    def trio(p, carry):
        for s in (0, 1, 2):
            g = p * 3 + s
            for j in range(GCH):
                gather(s, g * GCH + j, j).wait()
            store(s, g).start()
            o = (s + 2) % 3

            @pl.when(g >= 1)
            def _():
                store(o, g - 1).wait()

            @pl.when(g <= NGRP - 3)
            def _():
                fill(o, g + 2)

        return carry

    lax.fori_loop(0, NGRP // 3, trio, 0)

    g_last = NGRP - 1
    for j in range(GCH):
        gather(0, g_last * GCH + j, j).wait()
    store(0, g_last).start()
    store(2, g_last - 1).wait()
    store(0, g_last).wait()


@functools.partial(
    pl.kernel,
    out_type=jax.ShapeDtypeStruct((NTOK, DIM), jnp.float32),
    mesh=plsc.VectorSubcoreMesh(
        core_axis_name="c", subcore_axis_name="s",
        num_cores=NC, num_subcores=NS),
    scratch_types=[
        pltpu.VMEM((NCHUNK, CHUNK), jnp.int32),
        pltpu.VMEM((NSLAB, GCH * CHUNK, DIM), jnp.float32),
    ] + [pltpu.SemaphoreType.DMA] * 6,
)
def _gather_sc(idx_hbm, tab_hbm, out_hbm, idx_v, rows, g0, g1, g2, s0, s1, s2):
    _gather_sc_body(idx_hbm, tab_hbm, out_hbm, idx_v, rows, g0, g1, g2, s0, s1, s2)


def kernel(x, W_src, A, B):
    merged = _merge(W_src, A, B)
    idx = x.reshape(NW * NCHUNK, CHUNK).astype(jnp.int32)
    out = _gather_sc(idx, merged)
    return out.reshape(x.shape[0], x.shape[1], DIM)
